# baseline (speedup 1.0000x reference)
import functools
import math

import jax
import jax.numpy as jnp
from jax import lax
from jax.experimental import pallas as pl
from jax.experimental.pallas import tpu as pltpu
from jax.experimental.pallas import tpu_sc as plsc

D_MODEL = 64
SCALE = float(math.sqrt(D_MODEL))

NUM_CORES = 2
NUM_SUBCORES = 16
NUM_WORKERS = NUM_CORES * NUM_SUBCORES

BBLK = 128
OSTRIDE = 129


def _make_sc_lookup(b: int, t: int, d: int, vocab: int):
    assert b == NUM_WORKERS * BBLK and d == 64 and t % 4 == 0

    mesh = plsc.VectorSubcoreMesh(core_axis_name="c", subcore_axis_name="s")

    @functools.partial(
        pl.kernel,
        out_type=jax.ShapeDtypeStruct((t, d, b), jnp.float32),
        mesh=mesh,
        scratch_types=[
            pltpu.VMEM((t, BBLK), jnp.int32),
            pltpu.VMEM((4, BBLK, 128), jnp.float32),
            pltpu.VMEM((2, d, OSTRIDE), jnp.float32),
        ] + [pltpu.SemaphoreType.DMA] * 6,
        compiler_params=pltpu.CompilerParams(needs_layout_passes=False),
    )
    def lookup(xt_hbm, lutp_hbm, out_hbm, idx_v, rows_v, obuf_v, *sems):
        gsems = sems[:4]
        osems = sems[4:]

        w = lax.axis_index("s") * NUM_CORES + lax.axis_index("c")
        ob = pl.ds(w * BBLK, BBLK)
        pltpu.sync_copy(xt_hbm.at[:, ob], idx_v)

        lane = jax.lax.iota(jnp.int32, 16)
        kc_rows = [lane + kc * 16 for kc in range(d // 16)]

        for par in range(4):
            pltpu.async_copy(lutp_hbm.at[idx_v.at[par]],
                             rows_v.at[par], gsems[par])

        def gg_body(gg, carry):
            for par in range(4):
                g = gg * 4 + par
                pltpu.make_async_copy(lutp_hbm.at[idx_v.at[g]],
                                      rows_v.at[par], gsems[par]).wait()

                obi = par % 2
                @pl.when(g >= 2)
                def _wait_out():
                    pltpu.make_async_copy(
                        obuf_v.at[obi, :, pl.ds(0, BBLK)],
                        out_hbm.at[g - 2, :, ob], osems[obi]).wait()

                @plsc.parallel_loop(0, BBLK, 1, unroll=4)
                def j_body(j):
                    jv = jnp.full((16,), 0, jnp.int32) + j
                    for kc in range(d // 16):
                        v = rows_v[par, j, pl.ds(kc * 16, 16)] * SCALE
                        plsc.store_scatter(obuf_v.at[obi],
                                           [kc_rows[kc], jv], v)

                pltpu.async_copy(obuf_v.at[obi, :, pl.ds(0, BBLK)],
                                 out_hbm.at[g, :, ob], osems[obi])

                @pl.when(gg < t // 4 - 1)
                def _next_gather():
                    pltpu.async_copy(lutp_hbm.at[idx_v.at[g + 4]],
                                     rows_v.at[par], gsems[par])
            return carry

        lax.fori_loop(0, t // 4, gg_body, 0)

        for obi, gl in ((0, t - 2), (1, t - 1)):
            pltpu.make_async_copy(obuf_v.at[obi, :, pl.ds(0, BBLK)],
                                  out_hbm.at[gl, :, ob], osems[obi]).wait()

    return lookup


def kernel(x, lut):
    b, t = x.shape
    vocab, d = lut.shape
    xt = x.T.astype(jnp.int32)
    lutp = jnp.pad(lut, ((0, 0), (0, 128 - d)))
    out = _make_sc_lookup(b, t, d, vocab)(xt, lutp)
    return jnp.transpose(out, (2, 0, 1))


# speedup vs baseline: 1.0075x; 1.0075x over previous
import functools
import math

import jax
import jax.numpy as jnp
from jax import lax
from jax.experimental import pallas as pl
from jax.experimental.pallas import tpu as pltpu
from jax.experimental.pallas import tpu_sc as plsc

D_MODEL = 64
SCALE = float(math.sqrt(D_MODEL))

NUM_CORES = 2
NUM_SUBCORES = 16
NUM_WORKERS = NUM_CORES * NUM_SUBCORES

BBLK = 128
OSTRIDE = 128


def _make_sc_lookup(b: int, t: int, d: int, vocab: int):
    assert b == NUM_WORKERS * BBLK and d == 64 and t % 4 == 0

    mesh = plsc.VectorSubcoreMesh(core_axis_name="c", subcore_axis_name="s")

    @functools.partial(
        pl.kernel,
        out_type=jax.ShapeDtypeStruct((t, d, b), jnp.float32),
        mesh=mesh,
        scratch_types=[
            pltpu.VMEM((t, BBLK), jnp.int32),
            pltpu.VMEM((4, BBLK, 128), jnp.float32),
            pltpu.VMEM((2, d, OSTRIDE), jnp.float32),
        ] + [pltpu.SemaphoreType.DMA] * 6,
        compiler_params=pltpu.CompilerParams(needs_layout_passes=False),
    )
    def lookup(xt_hbm, lutp_hbm, out_hbm, idx_v, rows_v, obuf_v, *sems):
        gsems = sems[:4]
        osems = sems[4:]

        w = lax.axis_index("s") * NUM_CORES + lax.axis_index("c")
        ob = pl.ds(w * BBLK, BBLK)
        pltpu.sync_copy(xt_hbm.at[:, ob], idx_v)

        lane = jax.lax.iota(jnp.int32, 16)
        kc_rows = [lane + kc * 16 for kc in range(d // 16)]

        for par in range(4):
            pltpu.async_copy(lutp_hbm.at[idx_v.at[par]],
                             rows_v.at[par], gsems[par])

        def gg_body(gg, carry):
            for par in range(4):
                g = gg * 4 + par
                pltpu.make_async_copy(lutp_hbm.at[idx_v.at[g]],
                                      rows_v.at[par], gsems[par]).wait()

                obi = par % 2
                @pl.when(g >= 2)
                def _wait_out():
                    pltpu.make_async_copy(
                        obuf_v.at[obi, :, pl.ds(0, BBLK)],
                        out_hbm.at[g - 2, :, ob], osems[obi]).wait()

                @plsc.parallel_loop(0, BBLK, 1, unroll=4)
                def j_body(j):
                    jv = jnp.full((16,), 0, jnp.int32) + j
                    for kc in range(d // 16):
                        v = rows_v[par, j, pl.ds(kc * 16, 16)] * SCALE
                        plsc.store_scatter(obuf_v.at[obi],
                                           [kc_rows[kc], jv], v)

                pltpu.async_copy(obuf_v.at[obi, :, pl.ds(0, BBLK)],
                                 out_hbm.at[g, :, ob], osems[obi])

                @pl.when(gg < t // 4 - 1)
                def _next_gather():
                    pltpu.async_copy(lutp_hbm.at[idx_v.at[g + 4]],
                                     rows_v.at[par], gsems[par])
            return carry

        lax.fori_loop(0, t // 4, gg_body, 0)

        for obi, gl in ((0, t - 2), (1, t - 1)):
            pltpu.make_async_copy(obuf_v.at[obi, :, pl.ds(0, BBLK)],
                                  out_hbm.at[gl, :, ob], osems[obi]).wait()

    return lookup


def kernel(x, lut):
    b, t = x.shape
    vocab, d = lut.shape
    xt = x.T.astype(jnp.int32)
    lutp = jnp.pad(lut, ((0, 0), (0, 128 - d)))
    out = _make_sc_lookup(b, t, d, vocab)(xt, lutp)
    return jnp.transpose(out, (2, 0, 1))
